# baseline (speedup 1.0000x reference)
import jax
import jax.numpy as jnp
from jax import lax
from jax.experimental import pallas as pl
from jax.experimental.pallas import tpu as pltpu
from jax.experimental.pallas import tpu_sc as plsc

f32 = jnp.float32

N, E, D, H, DE = 10000, 320000, 128, 128, 16
NC, NS = 2, 16
NW = NC * NS
EW = E // NW
TB = 400
NB = EW // TB
RPT = 624
RTL = N - RPT * NS
TD = 200
ND = EW // TD


def _silu(x):
    return x * jax.nn.sigmoid(x)


def _prep_body(h_ref, wa_ref, wb_ref, hw1_ref, hw2_ref):
    hblk = h_ref[...]
    hw1_ref[...] = jnp.dot(hblk, wa_ref[...], preferred_element_type=f32)
    hw2_ref[...] = jnp.dot(hblk, wb_ref[...], preferred_element_type=f32)


def _prep(h, wa, wb):
    TN = 2000
    return pl.pallas_call(
        _prep_body,
        grid=(N // TN,),
        in_specs=[pl.BlockSpec((TN, D), lambda i: (i, 0)),
                  pl.BlockSpec((D, H), lambda i: (0, 0)),
                  pl.BlockSpec((D, H), lambda i: (0, 0))],
        out_specs=[pl.BlockSpec((TN, H), lambda i: (i, 0)),
                   pl.BlockSpec((TN, H), lambda i: (i, 0))],
        out_shape=[jax.ShapeDtypeStruct((N, H), f32),
                   jax.ShapeDtypeStruct((N, H), f32)],
    )(h, wa, wb)


def _gather1_body(row_hbm, col_hbm, hw1_hbm, hw2_hbm,
                  xa_hbm, xb_hbm,
                  ir_v, ic_v, xa_v, xb_v, s0, s1):
    wid = lax.axis_index("s") * NC + lax.axis_index("c")

    def step(i, carry):
        base = pl.multiple_of(wid * EW + i * TB, 8)
        pltpu.sync_copy(row_hbm.at[pl.ds(base, TB)], ir_v)
        pltpu.sync_copy(col_hbm.at[pl.ds(base, TB)], ic_v)
        c0 = pltpu.async_copy(hw1_hbm.at[ir_v], xa_v, s0)
        c1 = pltpu.async_copy(hw2_hbm.at[ic_v], xb_v, s1)
        c0.wait()
        c1.wait()
        pltpu.sync_copy(xa_v, xa_hbm.at[pl.ds(base, TB)])
        pltpu.sync_copy(xb_v, xb_hbm.at[pl.ds(base, TB)])
        return carry

    lax.fori_loop(0, NB, step, 0)


def _gather1(row, col, hw1, hw2):
    mesh = plsc.VectorSubcoreMesh(core_axis_name="c", subcore_axis_name="s")
    return pl.kernel(
        _gather1_body,
        out_type=[jax.ShapeDtypeStruct((E, H), f32),
                  jax.ShapeDtypeStruct((E, H), f32)],
        mesh=mesh,
        scratch_types=[pltpu.VMEM((TB,), jnp.int32),
                       pltpu.VMEM((TB,), jnp.int32),
                       pltpu.VMEM((TB, H), f32),
                       pltpu.VMEM((TB, H), f32),
                       pltpu.SemaphoreType.DMA,
                       pltpu.SemaphoreType.DMA],
    )(row, col, hw1, hw2)


def _gather2_body(row_hbm, col_hbm, cp_hbm,
                  cr_hbm, cc_hbm,
                  ir_v, ic_v, cr_v, cc_v, s0, s1):
    wid = lax.axis_index("s") * NC + lax.axis_index("c")

    def step(i, carry):
        base = pl.multiple_of(wid * EW + i * TB, 8)
        pltpu.sync_copy(row_hbm.at[pl.ds(base, TB)], ir_v)
        pltpu.sync_copy(col_hbm.at[pl.ds(base, TB)], ic_v)
        c0 = pltpu.async_copy(cp_hbm.at[ir_v], cr_v, s0)
        c1 = pltpu.async_copy(cp_hbm.at[ic_v], cc_v, s1)
        c0.wait()
        c1.wait()
        pltpu.sync_copy(cr_v, cr_hbm.at[pl.ds(base, TB)])
        pltpu.sync_copy(cc_v, cc_hbm.at[pl.ds(base, TB)])
        return carry

    lax.fori_loop(0, NB, step, 0)


def _gather2(row, col, cpad):
    mesh = plsc.VectorSubcoreMesh(core_axis_name="c", subcore_axis_name="s")
    return pl.kernel(
        _gather2_body,
        compiler_params=pltpu.CompilerParams(use_tc_tiling_on_sc=False),
        out_type=[jax.ShapeDtypeStruct((E, 16), f32),
                  jax.ShapeDtypeStruct((E, 16), f32)],
        mesh=mesh,
        scratch_types=[pltpu.VMEM((TB,), jnp.int32),
                       pltpu.VMEM((TB,), jnp.int32),
                       pltpu.VMEM((TB, 16), f32),
                       pltpu.VMEM((TB, 16), f32),
                       pltpu.SemaphoreType.DMA,
                       pltpu.SemaphoreType.DMA],
    )(row, col, cpad)


TEDGE = 2000


def _edge_body(xa_ref, xb_ref, cr_ref, cc_ref, ea_ref,
               wea_ref, wr_ref, b1_ref, w2_ref, b2_ref,
               wc1_ref, bc1_ref, wc2_ref, bc2_ref,
               ef_ref, t16_ref):
    cd = cr_ref[...] - cc_ref[...]
    radial = jnp.sum(cd * cd, axis=1, keepdims=True)
    bf16 = jnp.bfloat16
    x = (xa_ref[...] + xb_ref[...]
         + radial * wr_ref[...]
         + jnp.dot(ea_ref[...], wea_ref[...], preferred_element_type=f32)
         + b1_ref[...])
    m = _silu(x)
    y = jnp.dot(m.astype(bf16), w2_ref[...].astype(bf16),
                preferred_element_type=f32) + b2_ref[...]
    ef = _silu(y)
    z = jnp.dot(ef.astype(bf16), wc1_ref[...].astype(bf16),
                preferred_element_type=f32) + bc1_ref[...]
    ch = _silu(z)
    scale = jnp.sum(ch * wc2_ref[...], axis=1, keepdims=True) + bc2_ref[...]
    lane = lax.broadcasted_iota(jnp.int32, (TEDGE, 16), 1)
    ef_ref[...] = ef
    t16_ref[...] = cd * scale + (lane == 3).astype(f32)


def _edge(xa, xb, cr, cc, ea, wea, wr, b1, w2, b2, wc1, bc1, wc2, bc2):
    wfull = lambda shape: pl.BlockSpec(shape, lambda i: (0, 0))
    return pl.pallas_call(
        _edge_body,
        grid=(E // TEDGE,),
        in_specs=[pl.BlockSpec((TEDGE, H), lambda i: (i, 0)),
                  pl.BlockSpec((TEDGE, H), lambda i: (i, 0)),
                  pl.BlockSpec((TEDGE, 16), lambda i: (i, 0)),
                  pl.BlockSpec((TEDGE, 16), lambda i: (i, 0)),
                  pl.BlockSpec((TEDGE, DE), lambda i: (i, 0)),
                  wfull((DE, H)), wfull((1, H)), wfull((1, H)),
                  wfull((H, H)), wfull((1, H)),
                  wfull((H, H)), wfull((1, H)),
                  wfull((1, H)), wfull((1, 1))],
        out_specs=[pl.BlockSpec((TEDGE, H), lambda i: (i, 0)),
                   pl.BlockSpec((TEDGE, 16), lambda i: (i, 0))],
        out_shape=[jax.ShapeDtypeStruct((E, H), f32),
                   jax.ShapeDtypeStruct((E, 16), f32)],
    )(xa, xb, cr, cc, ea, wea, wr, b1, w2, b2, wc1, bc1, wc2, bc2)


def _scatter_body(row_hbm, ef_hbm, t16_hbm, z128_hbm, z16_hbm,
                  pagg_hbm, pt16_hbm,
                  agg_sh, t_sh, idx_v, ef_v, t_v, s0, s1, s2):
    cid = lax.axis_index("c")
    sid = lax.axis_index("s")
    r0 = pl.multiple_of(sid * RPT, 8)
    pltpu.sync_copy(z128_hbm.at[pl.ds(r0, RPT)], agg_sh.at[pl.ds(r0, RPT)])
    pltpu.sync_copy(z16_hbm.at[pl.ds(r0, RPT)], t_sh.at[pl.ds(r0, RPT)])

    @pl.when(sid == NS - 1)
    def _():
        t0 = RPT * NS
        pltpu.sync_copy(z128_hbm.at[pl.ds(t0, RTL)], agg_sh.at[pl.ds(t0, RTL)])
        pltpu.sync_copy(z16_hbm.at[pl.ds(t0, RTL)], t_sh.at[pl.ds(t0, RTL)])

    plsc.subcore_barrier()

    def step(i, carry):
        base = pl.multiple_of((cid * NS + sid) * EW + i * TD, 8)
        pltpu.sync_copy(row_hbm.at[pl.ds(base, TD)], idx_v)
        c0 = pltpu.async_copy(ef_hbm.at[pl.ds(base, TD)], ef_v, s0)
        c1 = pltpu.async_copy(t16_hbm.at[pl.ds(base, TD)], t_v, s1)
        c0.wait()
        c1.wait()
        pltpu.sync_copy(ef_v, agg_sh.at[idx_v], add=True)
        pltpu.sync_copy(t_v, t_sh.at[idx_v], add=True)
        return carry

    lax.fori_loop(0, ND, step, 0)
    plsc.subcore_barrier()
    pltpu.sync_copy(agg_sh.at[pl.ds(r0, RPT)], pagg_hbm.at[cid, pl.ds(r0, RPT)])
    pltpu.sync_copy(t_sh.at[pl.ds(r0, RPT)], pt16_hbm.at[cid, pl.ds(r0, RPT)])

    @pl.when(sid == NS - 1)
    def _():
        t0 = RPT * NS
        pltpu.sync_copy(agg_sh.at[pl.ds(t0, RTL)], pagg_hbm.at[cid, pl.ds(t0, RTL)])
        pltpu.sync_copy(t_sh.at[pl.ds(t0, RTL)], pt16_hbm.at[cid, pl.ds(t0, RTL)])


def _scatter(row, ef, t16, z128, z16):
    mesh = plsc.VectorSubcoreMesh(core_axis_name="c", subcore_axis_name="s")
    return pl.kernel(
        _scatter_body,
        compiler_params=pltpu.CompilerParams(use_tc_tiling_on_sc=False),
        out_type=[jax.ShapeDtypeStruct((NC, N, H), f32),
                  jax.ShapeDtypeStruct((NC, N, 16), f32)],
        mesh=mesh,
        scratch_types=[pltpu.VMEM_SHARED((N, H), f32),
                       pltpu.VMEM_SHARED((N, 16), f32),
                       pltpu.VMEM((TD,), jnp.int32),
                       pltpu.VMEM((TD, H), f32),
                       pltpu.VMEM((TD, 16), f32),
                       pltpu.SemaphoreType.DMA,
                       pltpu.SemaphoreType.DMA,
                       pltpu.SemaphoreType.DMA],
    )(row, ef, t16, z128, z16)


TNODE = 2000


def _node_body(h_ref, pa0_ref, pa1_ref, pt0_ref, pt1_ref, cp_ref,
               wna_ref, wnb_ref, b1_ref, w2_ref, b2_ref,
               ho_ref, cn_ref):
    agg = pa0_ref[...] + pa1_ref[...]
    t16 = pt0_ref[...] + pt1_ref[...]
    cnt = jnp.clip(t16[:, 3:4], 1.0, None)
    lane = lax.broadcasted_iota(jnp.int32, (TNODE, 16), 1)
    cn_ref[...] = cp_ref[...] + jnp.where(lane < 3, t16 / cnt, 0.0)
    x = (jnp.dot(h_ref[...], wna_ref[...], preferred_element_type=f32)
         + jnp.dot(agg, wnb_ref[...], preferred_element_type=f32)
         + b1_ref[...])
    nh = _silu(x)
    ho_ref[...] = jnp.dot(nh, w2_ref[...], preferred_element_type=f32) + b2_ref[...]


def _node(h, pa0, pa1, pt0, pt1, cpad, wna, wnb, b1, w2, b2):
    wfull = lambda shape: pl.BlockSpec(shape, lambda i: (0, 0))
    return pl.pallas_call(
        _node_body,
        grid=(N // TNODE,),
        in_specs=[pl.BlockSpec((TNODE, D), lambda i: (i, 0)),
                  pl.BlockSpec((TNODE, H), lambda i: (i, 0)),
                  pl.BlockSpec((TNODE, H), lambda i: (i, 0)),
                  pl.BlockSpec((TNODE, 16), lambda i: (i, 0)),
                  pl.BlockSpec((TNODE, 16), lambda i: (i, 0)),
                  pl.BlockSpec((TNODE, 16), lambda i: (i, 0)),
                  wfull((D, H)), wfull((H, H)), wfull((1, H)),
                  wfull((H, D)), wfull((1, D))],
        out_specs=[pl.BlockSpec((TNODE, D), lambda i: (i, 0)),
                   pl.BlockSpec((TNODE, 16), lambda i: (i, 0))],
        out_shape=[jax.ShapeDtypeStruct((N, D), f32),
                   jax.ShapeDtypeStruct((N, 16), f32)],
    )(h, pa0, pa1, pt0, pt1, cpad, wna, wnb, b1, w2, b2)


def kernel(h, edge_index, coord, edge_attr,
           We1, be1, We2, be2, Wn1, bn1, Wn2, bn2, Wc1, bc1, Wc2, bc2):
    row = edge_index[0]
    col = edge_index[1]
    cpad = jnp.pad(coord, ((0, 0), (0, 13)))

    hw1, hw2 = _prep(h, We1[:D], We1[D:2 * D])
    xa, xb = _gather1(row, col, hw1, hw2)
    cr, cc = _gather2(row, col, cpad)

    ef, t16 = _edge(xa, xb, cr, cc, edge_attr,
                    We1[2 * D + 1:], We1[2 * D].reshape(1, H),
                    be1.reshape(1, H), We2, be2.reshape(1, H),
                    Wc1, bc1.reshape(1, H), Wc2.reshape(1, H),
                    bc2.reshape(1, 1))

    z128 = jnp.zeros((N, H), f32)
    z16 = jnp.zeros((N, 16), f32)
    pagg, pt16 = _scatter(row, ef, t16, z128, z16)

    ho, cn = _node(h, pagg[0], pagg[1], pt16[0], pt16[1], cpad,
                   Wn1[:D], Wn1[D:], bn1.reshape(1, H), Wn2,
                   bn2.reshape(1, D))
    return (ho, cn[:, :3], edge_attr)


# speedup vs baseline: 1.2265x; 1.2265x over previous
import jax
import jax.numpy as jnp
from jax import lax
from jax.experimental import pallas as pl
from jax.experimental.pallas import tpu as pltpu
from jax.experimental.pallas import tpu_sc as plsc

f32 = jnp.float32

N, E, D, H, DE = 10000, 320000, 128, 128, 16
NC, NS = 2, 16
NW = NC * NS
EW = E // NW
TB = 200
NB = EW // TB
RPT = 624
RTL = N - RPT * NS
TD = 200
ND = EW // TD


def _silu(x):
    return x * jax.nn.sigmoid(x)


def _prep_body(h_ref, wa_ref, wb_ref, hw1_ref, hw2_ref):
    hblk = h_ref[...]
    hw1_ref[...] = jnp.dot(hblk, wa_ref[...], preferred_element_type=f32)
    hw2_ref[...] = jnp.dot(hblk, wb_ref[...], preferred_element_type=f32)


def _prep(h, wa, wb):
    TN = 2000
    return pl.pallas_call(
        _prep_body,
        grid=(N // TN,),
        in_specs=[pl.BlockSpec((TN, D), lambda i: (i, 0)),
                  pl.BlockSpec((D, H), lambda i: (0, 0)),
                  pl.BlockSpec((D, H), lambda i: (0, 0))],
        out_specs=[pl.BlockSpec((TN, H), lambda i: (i, 0)),
                   pl.BlockSpec((TN, H), lambda i: (i, 0))],
        out_shape=[jax.ShapeDtypeStruct((N, H), f32),
                   jax.ShapeDtypeStruct((N, H), f32)],
    )(h, wa, wb)


def _gather_body(row_hbm, col_hbm, hw1_hbm, hw2_hbm, cp_hbm,
                 xa_hbm, xb_hbm, cr_hbm, cc_hbm,
                 ir0, ic0, xa0, xb0, cr0, cc0,
                 ir1, ic1, xa1, xb1, cr1, cc1,
                 sa0, sb0, sr0, sc0, sa1, sb1, sr1, sc1):
    wid = lax.axis_index("s") * NC + lax.axis_index("c")
    bufs = ((ir0, ic0, xa0, xb0, cr0, cc0, sa0, sb0, sr0, sc0),
            (ir1, ic1, xa1, xb1, cr1, cc1, sa1, sb1, sr1, sc1))

    def issue(i, buf):
        (ir, ic, xa, xb, cr, cc, sa, sb, sr, sc) = buf
        base = pl.multiple_of(wid * EW + i * TB, 8)
        pltpu.sync_copy(row_hbm.at[pl.ds(base, TB)], ir)
        pltpu.sync_copy(col_hbm.at[pl.ds(base, TB)], ic)
        pltpu.async_copy(hw1_hbm.at[ir], xa, sa)
        pltpu.async_copy(hw2_hbm.at[ic], xb, sb)
        pltpu.async_copy(cp_hbm.at[ir], cr, sr)
        pltpu.async_copy(cp_hbm.at[ic], cc, sc)

    def drain(i, buf):
        (ir, ic, xa, xb, cr, cc, sa, sb, sr, sc) = buf
        base = pl.multiple_of(wid * EW + i * TB, 8)
        pltpu.make_async_copy(hw1_hbm.at[ir], xa, sa).wait()
        pltpu.make_async_copy(hw2_hbm.at[ic], xb, sb).wait()
        pltpu.make_async_copy(cp_hbm.at[ir], cr, sr).wait()
        pltpu.make_async_copy(cp_hbm.at[ic], cc, sc).wait()
        pltpu.sync_copy(xa, xa_hbm.at[pl.ds(base, TB)])
        pltpu.sync_copy(xb, xb_hbm.at[pl.ds(base, TB)])
        pltpu.sync_copy(cr, cr_hbm.at[pl.ds(base, TB)])
        pltpu.sync_copy(cc, cc_hbm.at[pl.ds(base, TB)])

    issue(0, bufs[0])

    def pair(g, carry):
        issue(2 * g + 1, bufs[1])
        drain(2 * g, bufs[0])
        issue(2 * g + 2, bufs[0])
        drain(2 * g + 1, bufs[1])
        return carry

    lax.fori_loop(0, NB // 2 - 1, pair, 0)
    issue(NB - 1, bufs[1])
    drain(NB - 2, bufs[0])
    drain(NB - 1, bufs[1])


def _gather(row, col, hw1, hw2, cpad):
    mesh = plsc.VectorSubcoreMesh(core_axis_name="c", subcore_axis_name="s")
    i32 = jnp.int32
    return pl.kernel(
        _gather_body,
        compiler_params=pltpu.CompilerParams(use_tc_tiling_on_sc=False),
        out_type=[jax.ShapeDtypeStruct((E, H), f32),
                  jax.ShapeDtypeStruct((E, H), f32),
                  jax.ShapeDtypeStruct((E, 16), f32),
                  jax.ShapeDtypeStruct((E, 16), f32)],
        mesh=mesh,
        scratch_types=[pltpu.VMEM((TB,), i32), pltpu.VMEM((TB,), i32),
                       pltpu.VMEM((TB, H), f32), pltpu.VMEM((TB, H), f32),
                       pltpu.VMEM((TB, 16), f32), pltpu.VMEM((TB, 16), f32),
                       pltpu.VMEM((TB,), i32), pltpu.VMEM((TB,), i32),
                       pltpu.VMEM((TB, H), f32), pltpu.VMEM((TB, H), f32),
                       pltpu.VMEM((TB, 16), f32), pltpu.VMEM((TB, 16), f32),
                       pltpu.SemaphoreType.DMA, pltpu.SemaphoreType.DMA,
                       pltpu.SemaphoreType.DMA, pltpu.SemaphoreType.DMA,
                       pltpu.SemaphoreType.DMA, pltpu.SemaphoreType.DMA,
                       pltpu.SemaphoreType.DMA, pltpu.SemaphoreType.DMA],
    )(row, col, hw1, hw2, cpad)


TEDGE = 2560


def _edge_body(xa_ref, xb_ref, cr_ref, cc_ref, eat_ref,
               wea_ref, wr_ref, b1_ref, w2_ref, b2_ref,
               wc1_ref, bc1_ref, wc2_ref, bc2_ref,
               ef_ref, t16_ref):
    cd = cr_ref[...] - cc_ref[...]
    radial = jnp.sum(cd * cd, axis=1, keepdims=True)
    x = (xa_ref[...] + xb_ref[...]
         + radial * wr_ref[...]
         + lax.dot_general(eat_ref[...], wea_ref[...],
                           (((0,), (0,)), ((), ())),
                           preferred_element_type=f32)
         + b1_ref[...])
    m = _silu(x)
    y = jnp.dot(m, w2_ref[...], preferred_element_type=f32) + b2_ref[...]
    ef = _silu(y)
    z = jnp.dot(ef, wc1_ref[...], preferred_element_type=f32) + bc1_ref[...]
    ch = _silu(z)
    scale = jnp.sum(ch * wc2_ref[...], axis=1, keepdims=True) + bc2_ref[...]
    lane = lax.broadcasted_iota(jnp.int32, (TEDGE, 16), 1)
    ef_ref[...] = ef
    t16_ref[...] = cd * scale + (lane == 3).astype(f32)


def _edge(xa, xb, cr, cc, eat, wea, wr, b1, w2, b2, wc1, bc1, wc2, bc2):
    wfull = lambda shape: pl.BlockSpec(shape, lambda i: (0, 0))
    return pl.pallas_call(
        _edge_body,
        grid=(E // TEDGE,),
        in_specs=[pl.BlockSpec((TEDGE, H), lambda i: (i, 0)),
                  pl.BlockSpec((TEDGE, H), lambda i: (i, 0)),
                  pl.BlockSpec((TEDGE, 16), lambda i: (i, 0)),
                  pl.BlockSpec((TEDGE, 16), lambda i: (i, 0)),
                  pl.BlockSpec((DE, TEDGE), lambda i: (0, i)),
                  wfull((DE, H)), wfull((1, H)), wfull((1, H)),
                  wfull((H, H)), wfull((1, H)),
                  wfull((H, H)), wfull((1, H)),
                  wfull((1, H)), wfull((1, 1))],
        out_specs=[pl.BlockSpec((TEDGE, H), lambda i: (i, 0)),
                   pl.BlockSpec((TEDGE, 16), lambda i: (i, 0))],
        out_shape=[jax.ShapeDtypeStruct((E, H), f32),
                   jax.ShapeDtypeStruct((E, 16), f32)],
    )(xa, xb, cr, cc, eat, wea, wr, b1, w2, b2, wc1, bc1, wc2, bc2)


def _scatter_body(row_hbm, ef_hbm, t16_hbm, z128_hbm, z16_hbm,
                  pagg_hbm, pt16_hbm,
                  agg_sh, t_sh, idx_v, ef_v, t_v, s0, s1):
    cid = lax.axis_index("c")
    sid = lax.axis_index("s")
    r0 = pl.multiple_of(sid * RPT, 8)
    pltpu.sync_copy(z128_hbm.at[pl.ds(r0, RPT)], agg_sh.at[pl.ds(r0, RPT)])
    pltpu.sync_copy(z16_hbm.at[pl.ds(r0, RPT)], t_sh.at[pl.ds(r0, RPT)])

    @pl.when(sid == NS - 1)
    def _():
        t0 = RPT * NS
        pltpu.sync_copy(z128_hbm.at[pl.ds(t0, RTL)], agg_sh.at[pl.ds(t0, RTL)])
        pltpu.sync_copy(z16_hbm.at[pl.ds(t0, RTL)], t_sh.at[pl.ds(t0, RTL)])

    plsc.subcore_barrier()

    def step(i, carry):
        base = pl.multiple_of((cid * NS + sid) * EW + i * TD, 8)
        pltpu.sync_copy(row_hbm.at[pl.ds(base, TD)], idx_v)
        c0 = pltpu.async_copy(ef_hbm.at[pl.ds(base, TD)], ef_v, s0)
        c1 = pltpu.async_copy(t16_hbm.at[pl.ds(base, TD)], t_v, s1)
        c0.wait()
        c1.wait()
        pltpu.sync_copy(ef_v, agg_sh.at[idx_v], add=True)
        pltpu.sync_copy(t_v, t_sh.at[idx_v], add=True)
        return carry

    lax.fori_loop(0, ND, step, 0)
    plsc.subcore_barrier()
    pltpu.sync_copy(agg_sh.at[pl.ds(r0, RPT)], pagg_hbm.at[cid, pl.ds(r0, RPT)])
    pltpu.sync_copy(t_sh.at[pl.ds(r0, RPT)], pt16_hbm.at[cid, pl.ds(r0, RPT)])

    @pl.when(sid == NS - 1)
    def _():
        t0 = RPT * NS
        pltpu.sync_copy(agg_sh.at[pl.ds(t0, RTL)], pagg_hbm.at[cid, pl.ds(t0, RTL)])
        pltpu.sync_copy(t_sh.at[pl.ds(t0, RTL)], pt16_hbm.at[cid, pl.ds(t0, RTL)])


def _scatter(row, ef, t16, z128, z16):
    mesh = plsc.VectorSubcoreMesh(core_axis_name="c", subcore_axis_name="s")
    return pl.kernel(
        _scatter_body,
        compiler_params=pltpu.CompilerParams(use_tc_tiling_on_sc=False),
        out_type=[jax.ShapeDtypeStruct((NC, N, H), f32),
                  jax.ShapeDtypeStruct((NC, N, 16), f32)],
        mesh=mesh,
        scratch_types=[pltpu.VMEM_SHARED((N, H), f32),
                       pltpu.VMEM_SHARED((N, 16), f32),
                       pltpu.VMEM((TD,), jnp.int32),
                       pltpu.VMEM((TD, H), f32),
                       pltpu.VMEM((TD, 16), f32),
                       pltpu.SemaphoreType.DMA,
                       pltpu.SemaphoreType.DMA],
    )(row, ef, t16, z128, z16)


TNODE = 2000


def _node_body(h_ref, pa0_ref, pa1_ref, pt0_ref, pt1_ref, cp_ref,
               wna_ref, wnb_ref, b1_ref, w2_ref, b2_ref,
               ho_ref, cn_ref):
    agg = pa0_ref[...] + pa1_ref[...]
    t16 = pt0_ref[...] + pt1_ref[...]
    cnt = jnp.clip(t16[:, 3:4], 1.0, None)
    lane = lax.broadcasted_iota(jnp.int32, (TNODE, 16), 1)
    cn_ref[...] = cp_ref[...] + jnp.where(lane < 3, t16 / cnt, 0.0)
    x = (jnp.dot(h_ref[...], wna_ref[...], preferred_element_type=f32)
         + jnp.dot(agg, wnb_ref[...], preferred_element_type=f32)
         + b1_ref[...])
    nh = _silu(x)
    ho_ref[...] = jnp.dot(nh, w2_ref[...], preferred_element_type=f32) + b2_ref[...]


def _node(h, pa0, pa1, pt0, pt1, cpad, wna, wnb, b1, w2, b2):
    wfull = lambda shape: pl.BlockSpec(shape, lambda i: (0, 0))
    return pl.pallas_call(
        _node_body,
        grid=(N // TNODE,),
        in_specs=[pl.BlockSpec((TNODE, D), lambda i: (i, 0)),
                  pl.BlockSpec((TNODE, H), lambda i: (i, 0)),
                  pl.BlockSpec((TNODE, H), lambda i: (i, 0)),
                  pl.BlockSpec((TNODE, 16), lambda i: (i, 0)),
                  pl.BlockSpec((TNODE, 16), lambda i: (i, 0)),
                  pl.BlockSpec((TNODE, 16), lambda i: (i, 0)),
                  wfull((D, H)), wfull((H, H)), wfull((1, H)),
                  wfull((H, D)), wfull((1, D))],
        out_specs=[pl.BlockSpec((TNODE, D), lambda i: (i, 0)),
                   pl.BlockSpec((TNODE, 16), lambda i: (i, 0))],
        out_shape=[jax.ShapeDtypeStruct((N, D), f32),
                   jax.ShapeDtypeStruct((N, 16), f32)],
    )(h, pa0, pa1, pt0, pt1, cpad, wna, wnb, b1, w2, b2)


def kernel(h, edge_index, coord, edge_attr,
           We1, be1, We2, be2, Wn1, bn1, Wn2, bn2, Wc1, bc1, Wc2, bc2):
    row = edge_index[0]
    col = edge_index[1]
    cpad = jnp.pad(coord, ((0, 0), (0, 13)))

    hw1, hw2 = _prep(h, We1[:D], We1[D:2 * D])
    xa, xb, cr, cc = _gather(row, col, hw1, hw2, cpad)

    ef, t16 = _edge(xa, xb, cr, cc, edge_attr.T,
                    We1[2 * D + 1:], We1[2 * D].reshape(1, H),
                    be1.reshape(1, H), We2, be2.reshape(1, H),
                    Wc1, bc1.reshape(1, H), Wc2.reshape(1, H),
                    bc2.reshape(1, 1))

    z128 = jnp.zeros((N, H), f32)
    z16 = jnp.zeros((N, 16), f32)
    pagg, pt16 = _scatter(row, ef, t16, z128, z16)

    ho, cn = _node(h, pagg[0], pagg[1], pt16[0], pt16[1], cpad,
                   Wn1[:D], Wn1[D:], bn1.reshape(1, H), Wn2,
                   bn2.reshape(1, D))
    return (ho, cn[:, :3], edge_attr)
